# baseline (speedup 1.0000x reference)
import jax
import jax.numpy as jnp
from jax.experimental import pallas as pl
from jax.experimental.pallas import tpu as pltpu

_P = 8732
_M = 50
_MP = 52
_B = 32
_ROWS = 72
_LANES = 128
_PPAD = _ROWS * _LANES
_THRESH = 0.5
_NEGPOS = 3
_UNROLL = 4


def _body(tgt_ref, pr_ref, lin_ref, loc_ref, conf_ref, out_ref,
          ov_ref, bt_ref, c0_ref, c1_ref, c2_ref, c3_ref, bp_ref):
    f32 = jnp.float32
    shp = (_B, _ROWS, _LANES)
    px1 = pr_ref[0][None]
    py1 = pr_ref[1][None]
    px2 = pr_ref[2][None]
    py2 = pr_ref[3][None]
    parea = pr_ref[4][None]

    ov_ref[...] = jnp.full(shp, -2.0, f32)
    bt_ref[...] = jnp.zeros(shp, jnp.int32)

    def pass1(i, carry):
        ov = ov_ref[...]
        bt = bt_ref[...]
        lin = lin_ref[...][None]
        for j in range(_UNROLL):
            t = _UNROLL * i + j
            tx1 = tgt_ref[t, 0]
            ty1 = tgt_ref[t, 1]
            tx2 = tgt_ref[t, 2]
            ty2 = tgt_ref[t, 3]
            ix = jnp.maximum(jnp.minimum(tx2, px2) - jnp.maximum(tx1, px1), 0.0)
            iy = jnp.maximum(jnp.minimum(ty2, py2) - jnp.maximum(ty1, py1), 0.0)
            inter = ix * iy
            tarea = tgt_ref[t, 5]
            union = jnp.maximum(tarea + parea - inter, 1e-10)
            iou = inter / union
            m = jnp.max(jnp.max(iou, axis=2, keepdims=True), axis=1,
                        keepdims=True)
            bp = jnp.min(jnp.min(jnp.where(iou == m, lin, jnp.int32(2**30)),
                                 axis=2, keepdims=True),
                         axis=1, keepdims=True)
            validb = tgt_ref[t, 4][:, :, 0:1] > 0.0
            bp = jnp.where(validb, bp, -1)
            bp_ref[t] = jnp.broadcast_to(bp[:, 0, :], (_B, _LANES))
            better = iou > ov
            ov = jnp.where(better, iou, ov)
            bt = jnp.where(better, t, bt)
        ov_ref[...] = ov
        bt_ref[...] = bt
        return carry

    jax.lax.fori_loop(0, _MP // _UNROLL, pass1, 0)

    def pass2(i, carry):
        ov = ov_ref[...]
        bt = bt_ref[...]
        lin = lin_ref[...][None]
        for j in range(_UNROLL):
            t = (_MP - 1) - (_UNROLL * i + j)
            match = lin == bp_ref[t][:, None, :]
            ov = jnp.where(match, 2.0, ov)
            bt = jnp.where(match, t, bt)
        ov_ref[...] = ov
        bt_ref[...] = bt
        return carry

    jax.lax.fori_loop(0, _MP // _UNROLL, pass2, 0)

    def gather(i, carry):
        bt = bt_ref[...]
        c0 = c0_ref[...]
        c1 = c1_ref[...]
        c2 = c2_ref[...]
        c3 = c3_ref[...]
        for j in range(_UNROLL):
            t = _UNROLL * i + j
            eq = bt == t
            c0 = jnp.where(eq, tgt_ref[t, 6], c0)
            c1 = jnp.where(eq, tgt_ref[t, 7], c1)
            c2 = jnp.where(eq, tgt_ref[t, 8], c2)
            c3 = jnp.where(eq, tgt_ref[t, 9], c3)
        c0_ref[...] = c0
        c1_ref[...] = c1
        c2_ref[...] = c2
        c3_ref[...] = c3
        return carry

    zero = jnp.zeros(shp, f32)
    c0_ref[...] = zero
    c1_ref[...] = zero
    c2_ref[...] = zero
    c3_ref[...] = zero
    jax.lax.fori_loop(0, _MP // _UNROLL, gather, 0)

    lane_valid = lin_ref[...][None] < _P
    pos = jnp.logical_and(ov_ref[...] >= _THRESH, lane_valid)

    pcx = pr_ref[5][None]
    pcy = pr_ref[6][None]
    ivw = pr_ref[7][None]
    ivh = pr_ref[8][None]
    lwp = pr_ref[9][None]
    lhp = pr_ref[10][None]
    gcx = (c0_ref[...] - pcx) * ivw
    gcy = (c1_ref[...] - pcy) * ivh
    gw = c2_ref[...] + lwp
    gh = c3_ref[...] + lhp

    def _sl1(d):
        ad = jnp.abs(d)
        return jnp.where(ad < 1.0, 0.5 * ad * ad, ad - 0.5)

    sl1 = (_sl1(loc_ref[0] - gcx) + _sl1(loc_ref[1] - gcy)
           + _sl1(loc_ref[2] - gw) + _sl1(loc_ref[3] - gh))
    loss_l = jnp.sum(jnp.sum(jnp.where(pos, sl1, 0.0), axis=2,
                             keepdims=True), axis=1, keepdims=True)

    ca = conf_ref[0]
    cb = conf_ref[1]
    mx = jnp.maximum(ca, cb)
    lse = mx + jnp.log(jnp.exp(ca - mx) + jnp.exp(cb - mx))
    gath = jnp.where(pos, cb, ca)
    lc = lse - gath
    lc_pos = jnp.sum(jnp.sum(jnp.where(pos, lc, 0.0), axis=2,
                             keepdims=True), axis=1, keepdims=True)

    mine = jnp.where(jnp.logical_or(pos, jnp.logical_not(lane_valid)), 0.0, lc)
    mine = jnp.maximum(mine, 0.0)
    bits = jnp.maximum(jax.lax.bitcast_convert_type(mine, jnp.int32), 0)

    npos = jnp.sum(jnp.sum(pos.astype(jnp.int32), axis=2, keepdims=True),
                   axis=1, keepdims=True)
    k = jnp.minimum(_NEGPOS * npos, _P - 1)

    def bstep(i, pfx):
        cand = pfx | jnp.left_shift(jnp.int32(1), 30 - i)
        cnt = jnp.sum(jnp.sum((bits >= cand).astype(jnp.int32), axis=2,
                             keepdims=True), axis=1, keepdims=True)
        return jnp.where(cnt >= k, cand, pfx)

    tbits = jax.lax.fori_loop(0, 31, bstep, jnp.zeros((_B, 1, 1), jnp.int32))
    gt = bits > tbits
    cnt_gt = jnp.sum(jnp.sum(gt.astype(jnp.int32), axis=2, keepdims=True),
                     axis=1, keepdims=True)
    sum_gt = jnp.sum(jnp.sum(jnp.where(gt, mine, 0.0), axis=2,
                             keepdims=True), axis=1, keepdims=True)
    tval = jax.lax.bitcast_convert_type(tbits, f32)
    topk = sum_gt + (k - cnt_gt).astype(f32) * tval
    loss_c = lc_pos + jnp.where(k > 0, topk, 0.0)

    lane = jax.lax.broadcasted_iota(jnp.int32, (_B, _LANES), 1)
    llb = jnp.broadcast_to(loss_l[:, :, 0], (_B, _LANES))
    lcb = jnp.broadcast_to(loss_c[:, :, 0], (_B, _LANES))
    npb = jnp.broadcast_to(npos[:, :, 0].astype(f32), (_B, _LANES))
    out_ref[...] = jnp.where(lane == 0, llb,
                             jnp.where(lane == 1, lcb,
                                       jnp.where(lane == 2, npb, 0.0)))


def kernel(loc_data, conf_data, priors, targets):
    B = loc_data.shape[0]
    w = priors[:, 2]
    h = priors[:, 3]
    pf1 = priors[:, 0] - w * 0.5
    pf2 = priors[:, 1] - h * 0.5
    pf3 = priors[:, 0] + w * 0.5
    pf4 = priors[:, 1] + h * 0.5
    parea = (pf3 - pf1) * (pf4 - pf2)
    ch = jnp.stack([
        pf1, pf2, pf3, pf4, parea,
        priors[:, 0], priors[:, 1],
        1.0 / (0.1 * w), 1.0 / (0.1 * h),
        -5.0 * jnp.log(w), -5.0 * jnp.log(h),
    ], axis=0)
    pad = _PPAD - _P
    padvals = jnp.array([2.0, 2.0, 2.0, 2.0, 0.0, 0.0, 0.0, 1.0, 1.0, 0.0,
                         0.0], dtype=jnp.float32)
    ch = jnp.concatenate([ch, jnp.broadcast_to(padvals[:, None], (11, pad))],
                         axis=1)
    ch = ch.reshape(11, _ROWS, _LANES)
    lin = jnp.arange(_PPAD, dtype=jnp.int32).reshape(_ROWS, _LANES)

    loc_r = jnp.pad(jnp.transpose(loc_data, (2, 0, 1)), ((0, 0), (0, 0), (0, pad)))
    loc_r = loc_r.reshape(4, B, _ROWS, _LANES)
    conf_r = jnp.pad(jnp.transpose(conf_data, (2, 0, 1)), ((0, 0), (0, 0), (0, pad)))
    conf_r = conf_r.reshape(2, B, _ROWS, _LANES)

    validf = (jnp.sum(targets, axis=2) > 0.0).astype(jnp.float32)
    boxes = jnp.where(validf[:, :, None] > 0, targets, 3.0)
    tarea = (boxes[:, :, 2] - boxes[:, :, 0]) * (boxes[:, :, 3] - boxes[:, :, 1])
    tcx = (boxes[:, :, 0] + boxes[:, :, 2]) * 0.5
    tcy = (boxes[:, :, 1] + boxes[:, :, 3]) * 0.5
    ltw = 5.0 * jnp.log(jnp.maximum(boxes[:, :, 2] - boxes[:, :, 0], 1e-6))
    lth = 5.0 * jnp.log(jnp.maximum(boxes[:, :, 3] - boxes[:, :, 1], 1e-6))
    tchan = jnp.concatenate(
        [boxes, validf[:, :, None], tarea[:, :, None], tcx[:, :, None],
         tcy[:, :, None], ltw[:, :, None], lth[:, :, None]], axis=2)
    padrow = jnp.array([3.0, 3.0, 3.0, 3.0, 0.0, 0.0, 3.0, 3.0,
                        5.0 * jnp.log(1e-6), 5.0 * jnp.log(1e-6)],
                       dtype=jnp.float32)
    tchan = jnp.concatenate(
        [tchan, jnp.broadcast_to(padrow[None, None, :], (B, _MP - _M, 10))],
        axis=1)
    tgt = jnp.broadcast_to(jnp.transpose(tchan, (1, 2, 0))[:, :, :, None, None],
                           (_MP, 10, B, 1, _LANES))

    out = pl.pallas_call(
        _body,
        out_shape=jax.ShapeDtypeStruct((B, _LANES), jnp.float32),
        scratch_shapes=[
            pltpu.VMEM((B, _ROWS, _LANES), jnp.float32),
            pltpu.VMEM((B, _ROWS, _LANES), jnp.int32),
            pltpu.VMEM((B, _ROWS, _LANES), jnp.float32),
            pltpu.VMEM((B, _ROWS, _LANES), jnp.float32),
            pltpu.VMEM((B, _ROWS, _LANES), jnp.float32),
            pltpu.VMEM((B, _ROWS, _LANES), jnp.float32),
            pltpu.VMEM((_MP, B, _LANES), jnp.int32),
        ],
    )(tgt, ch, lin, loc_r, conf_r)

    loss_l = jnp.sum(out[:, 0])
    loss_c = jnp.sum(out[:, 1])
    n = jnp.maximum(jnp.sum(out[:, 2]), 1.0)
    return loss_l / n, loss_c / n


# speedup vs baseline: 1.1601x; 1.1601x over previous
import jax
import jax.numpy as jnp
from jax.experimental import pallas as pl
from jax.experimental.pallas import tpu as pltpu

_P = 8732
_M = 50
_MP = 52
_B = 32
_ROWS = 72
_LANES = 128
_PPAD = _ROWS * _LANES
_THRESH = 0.5
_NEGPOS = 3
_UNROLL = 4


def _body(tgt_ref, pr_ref, lin_ref, loc_ref, conf_ref, out_ref,
          ov_ref, bt_ref, c0_ref, c1_ref, c2_ref, c3_ref, bp_ref):
    f32 = jnp.float32
    shp = (_B, _ROWS, _LANES)
    px1 = pr_ref[0][None]
    py1 = pr_ref[1][None]
    px2 = pr_ref[2][None]
    py2 = pr_ref[3][None]
    parea = pr_ref[4][None]

    ov_ref[...] = jnp.full(shp, -2.0, f32)
    bt_ref[...] = jnp.zeros(shp, jnp.int32)

    def pass1(i, carry):
        ov = ov_ref[...]
        bt = bt_ref[...]
        lin = lin_ref[...][None]
        for j in range(_UNROLL):
            t = _UNROLL * i + j
            tx1 = tgt_ref[t, 0][:, None, :]
            ty1 = tgt_ref[t, 1][:, None, :]
            tx2 = tgt_ref[t, 2][:, None, :]
            ty2 = tgt_ref[t, 3][:, None, :]
            ix = jnp.maximum(jnp.minimum(tx2, px2) - jnp.maximum(tx1, px1), 0.0)
            iy = jnp.maximum(jnp.minimum(ty2, py2) - jnp.maximum(ty1, py1), 0.0)
            inter = ix * iy
            tarea = tgt_ref[t, 5][:, None, :]
            union = jnp.maximum(tarea + parea - inter, 1e-10)
            iou = inter / union
            m = jnp.max(jnp.max(iou, axis=2, keepdims=True), axis=1,
                        keepdims=True)
            bp = jnp.min(jnp.min(jnp.where(iou == m, lin, jnp.int32(2**30)),
                                 axis=2, keepdims=True),
                         axis=1, keepdims=True)
            validb = tgt_ref[t, 4][:, None, 0:1] > 0.0
            bp = jnp.where(validb, bp, -1)
            bp_ref[t] = jnp.broadcast_to(bp[:, 0, :], (_B, _LANES))
            better = iou > ov
            ov = jnp.where(better, iou, ov)
            bt = jnp.where(better, t, bt)
        ov_ref[...] = ov
        bt_ref[...] = bt
        return carry

    jax.lax.fori_loop(0, _MP // _UNROLL, pass1, 0)

    def pass2(i, carry):
        ov = ov_ref[...]
        bt = bt_ref[...]
        lin = lin_ref[...][None]
        for j in range(_UNROLL):
            t = (_MP - 1) - (_UNROLL * i + j)
            match = lin == bp_ref[t][:, None, :]
            ov = jnp.where(match, 2.0, ov)
            bt = jnp.where(match, t, bt)
        ov_ref[...] = ov
        bt_ref[...] = bt
        return carry

    jax.lax.fori_loop(0, _MP // _UNROLL, pass2, 0)

    def gather(i, carry):
        bt = bt_ref[...]
        c0 = c0_ref[...]
        c1 = c1_ref[...]
        c2 = c2_ref[...]
        c3 = c3_ref[...]
        for j in range(_UNROLL):
            t = _UNROLL * i + j
            eq = bt == t
            c0 = jnp.where(eq, tgt_ref[t, 6][:, None, :], c0)
            c1 = jnp.where(eq, tgt_ref[t, 7][:, None, :], c1)
            c2 = jnp.where(eq, tgt_ref[t, 8][:, None, :], c2)
            c3 = jnp.where(eq, tgt_ref[t, 9][:, None, :], c3)
        c0_ref[...] = c0
        c1_ref[...] = c1
        c2_ref[...] = c2
        c3_ref[...] = c3
        return carry

    zero = jnp.zeros(shp, f32)
    c0_ref[...] = zero
    c1_ref[...] = zero
    c2_ref[...] = zero
    c3_ref[...] = zero
    jax.lax.fori_loop(0, _MP // _UNROLL, gather, 0)

    lane_valid = lin_ref[...][None] < _P
    pos = jnp.logical_and(ov_ref[...] >= _THRESH, lane_valid)

    pcx = pr_ref[5][None]
    pcy = pr_ref[6][None]
    ivw = pr_ref[7][None]
    ivh = pr_ref[8][None]
    lwp = pr_ref[9][None]
    lhp = pr_ref[10][None]
    gcx = (c0_ref[...] - pcx) * ivw
    gcy = (c1_ref[...] - pcy) * ivh
    gw = c2_ref[...] + lwp
    gh = c3_ref[...] + lhp

    def _sl1(d):
        ad = jnp.abs(d)
        return jnp.where(ad < 1.0, 0.5 * ad * ad, ad - 0.5)

    sl1 = (_sl1(loc_ref[0] - gcx) + _sl1(loc_ref[1] - gcy)
           + _sl1(loc_ref[2] - gw) + _sl1(loc_ref[3] - gh))
    loss_l = jnp.sum(jnp.where(pos, sl1, 0.0), axis=(1, 2), keepdims=True)

    ca = conf_ref[0]
    cb = conf_ref[1]
    mx = jnp.maximum(ca, cb)
    lse = mx + jnp.log(jnp.exp(ca - mx) + jnp.exp(cb - mx))
    gath = jnp.where(pos, cb, ca)
    lc = lse - gath
    lc_pos = jnp.sum(jnp.where(pos, lc, 0.0), axis=(1, 2), keepdims=True)

    mine = jnp.where(jnp.logical_or(pos, jnp.logical_not(lane_valid)), 0.0, lc)
    mine = jnp.maximum(mine, 0.0)
    bits = jnp.maximum(jax.lax.bitcast_convert_type(mine, jnp.int32), 0)

    npos = jnp.sum(pos.astype(jnp.int32), axis=(1, 2), keepdims=True)
    k = jnp.minimum(_NEGPOS * npos, _P - 1)

    def bstep(i, pfx):
        cand = pfx | jnp.left_shift(jnp.int32(1), 30 - i)
        cnt = jnp.sum((bits >= cand).astype(jnp.int32), axis=(1, 2),
                      keepdims=True)
        return jnp.where(cnt >= k, cand, pfx)

    tbits = jax.lax.fori_loop(0, 31, bstep, jnp.zeros((_B, 1, 1), jnp.int32))
    gt = bits > tbits
    cnt_gt = jnp.sum(gt.astype(jnp.int32), axis=(1, 2), keepdims=True)
    sum_gt = jnp.sum(jnp.where(gt, mine, 0.0), axis=(1, 2), keepdims=True)
    tval = jax.lax.bitcast_convert_type(tbits, f32)
    topk = sum_gt + (k - cnt_gt).astype(f32) * tval
    loss_c = lc_pos + jnp.where(k > 0, topk, 0.0)

    lane = jax.lax.broadcasted_iota(jnp.int32, (_B, _LANES), 1)
    llb = jnp.broadcast_to(loss_l[:, :, 0], (_B, _LANES))
    lcb = jnp.broadcast_to(loss_c[:, :, 0], (_B, _LANES))
    npb = jnp.broadcast_to(npos[:, :, 0].astype(f32), (_B, _LANES))
    out_ref[...] = jnp.where(lane == 0, llb,
                             jnp.where(lane == 1, lcb,
                                       jnp.where(lane == 2, npb, 0.0)))


def kernel(loc_data, conf_data, priors, targets):
    B = loc_data.shape[0]
    w = priors[:, 2]
    h = priors[:, 3]
    pf1 = priors[:, 0] - w * 0.5
    pf2 = priors[:, 1] - h * 0.5
    pf3 = priors[:, 0] + w * 0.5
    pf4 = priors[:, 1] + h * 0.5
    parea = (pf3 - pf1) * (pf4 - pf2)
    ch = jnp.stack([
        pf1, pf2, pf3, pf4, parea,
        priors[:, 0], priors[:, 1],
        1.0 / (0.1 * w), 1.0 / (0.1 * h),
        -5.0 * jnp.log(w), -5.0 * jnp.log(h),
    ], axis=0)
    pad = _PPAD - _P
    padvals = jnp.array([2.0, 2.0, 2.0, 2.0, 0.0, 0.0, 0.0, 1.0, 1.0, 0.0,
                         0.0], dtype=jnp.float32)
    ch = jnp.concatenate([ch, jnp.broadcast_to(padvals[:, None], (11, pad))],
                         axis=1)
    ch = ch.reshape(11, _ROWS, _LANES)
    lin = jnp.arange(_PPAD, dtype=jnp.int32).reshape(_ROWS, _LANES)

    loc_r = jnp.pad(jnp.transpose(loc_data, (2, 0, 1)), ((0, 0), (0, 0), (0, pad)))
    loc_r = loc_r.reshape(4, B, _ROWS, _LANES)
    conf_r = jnp.pad(jnp.transpose(conf_data, (2, 0, 1)), ((0, 0), (0, 0), (0, pad)))
    conf_r = conf_r.reshape(2, B, _ROWS, _LANES)

    validf = (jnp.sum(targets, axis=2) > 0.0).astype(jnp.float32)
    boxes = jnp.where(validf[:, :, None] > 0, targets, 3.0)
    tarea = (boxes[:, :, 2] - boxes[:, :, 0]) * (boxes[:, :, 3] - boxes[:, :, 1])
    tcx = (boxes[:, :, 0] + boxes[:, :, 2]) * 0.5
    tcy = (boxes[:, :, 1] + boxes[:, :, 3]) * 0.5
    ltw = 5.0 * jnp.log(jnp.maximum(boxes[:, :, 2] - boxes[:, :, 0], 1e-6))
    lth = 5.0 * jnp.log(jnp.maximum(boxes[:, :, 3] - boxes[:, :, 1], 1e-6))
    tchan = jnp.concatenate(
        [boxes, validf[:, :, None], tarea[:, :, None], tcx[:, :, None],
         tcy[:, :, None], ltw[:, :, None], lth[:, :, None]], axis=2)
    padrow = jnp.array([3.0, 3.0, 3.0, 3.0, 0.0, 0.0, 3.0, 3.0,
                        5.0 * jnp.log(1e-6), 5.0 * jnp.log(1e-6)],
                       dtype=jnp.float32)
    tchan = jnp.concatenate(
        [tchan, jnp.broadcast_to(padrow[None, None, :], (B, _MP - _M, 10))],
        axis=1)
    tgt = jnp.broadcast_to(jnp.transpose(tchan, (1, 2, 0))[:, :, :, None],
                           (_MP, 10, B, _LANES))

    out = pl.pallas_call(
        _body,
        out_shape=jax.ShapeDtypeStruct((B, _LANES), jnp.float32),
        scratch_shapes=[
            pltpu.VMEM((B, _ROWS, _LANES), jnp.float32),
            pltpu.VMEM((B, _ROWS, _LANES), jnp.int32),
            pltpu.VMEM((B, _ROWS, _LANES), jnp.float32),
            pltpu.VMEM((B, _ROWS, _LANES), jnp.float32),
            pltpu.VMEM((B, _ROWS, _LANES), jnp.float32),
            pltpu.VMEM((B, _ROWS, _LANES), jnp.float32),
            pltpu.VMEM((_MP, B, _LANES), jnp.int32),
        ],
    )(tgt, ch, lin, loc_r, conf_r)

    loss_l = jnp.sum(out[:, 0])
    loss_c = jnp.sum(out[:, 1])
    n = jnp.maximum(jnp.sum(out[:, 2]), 1.0)
    return loss_l / n, loss_c / n


# speedup vs baseline: 1.2685x; 1.0934x over previous
import jax
import jax.numpy as jnp
from jax.experimental import pallas as pl
from jax.experimental.pallas import tpu as pltpu

_P = 8732
_M = 50
_MP = 56
_B = 32
_ROWS = 72
_LANES = 128
_PPAD = _ROWS * _LANES
_THRESH = 0.5
_NEGPOS = 3
_UNROLL = 8


def _body(tgt_ref, pr_ref, lin_ref, loc_ref, conf_ref, out_ref,
          ov_ref, bt_ref, c0_ref, c1_ref, c2_ref, c3_ref, bp_ref):
    f32 = jnp.float32
    shp = (_B, _ROWS, _LANES)
    px1 = pr_ref[0][None]
    py1 = pr_ref[1][None]
    px2 = pr_ref[2][None]
    py2 = pr_ref[3][None]
    parea = pr_ref[4][None]

    ov_ref[...] = jnp.full(shp, -2.0, f32)
    bt_ref[...] = jnp.zeros(shp, jnp.int32)

    def pass1(i, carry):
        ov = ov_ref[...]
        bt = bt_ref[...]
        lin = lin_ref[...][None]
        for j in range(_UNROLL):
            t = _UNROLL * i + j
            tx1 = tgt_ref[t, 0][:, None, :]
            ty1 = tgt_ref[t, 1][:, None, :]
            tx2 = tgt_ref[t, 2][:, None, :]
            ty2 = tgt_ref[t, 3][:, None, :]
            ix = jnp.maximum(jnp.minimum(tx2, px2) - jnp.maximum(tx1, px1), 0.0)
            iy = jnp.maximum(jnp.minimum(ty2, py2) - jnp.maximum(ty1, py1), 0.0)
            inter = ix * iy
            tarea = tgt_ref[t, 5][:, None, :]
            union = jnp.maximum(tarea + parea - inter, 1e-10)
            iou = inter / union
            m = jnp.max(iou, axis=(1, 2), keepdims=True)
            bp = jnp.min(jnp.where(iou == m, lin, jnp.int32(2**30)),
                         axis=(1, 2), keepdims=True)
            validb = tgt_ref[t, 4][:, None, 0:1] > 0.0
            bp = jnp.where(validb, bp, -1)
            bp_ref[t] = jnp.broadcast_to(bp[:, 0, :], (_B, _LANES))
            better = iou > ov
            ov = jnp.where(better, iou, ov)
            bt = jnp.where(better, t, bt)
        ov_ref[...] = ov
        bt_ref[...] = bt
        return carry

    jax.lax.fori_loop(0, _MP // _UNROLL, pass1, 0)

    def pass2(i, carry):
        ov = ov_ref[...]
        bt = bt_ref[...]
        lin = lin_ref[...][None]
        for j in range(_UNROLL):
            t = (_MP - 1) - (_UNROLL * i + j)
            match = lin == bp_ref[t][:, None, :]
            ov = jnp.where(match, 2.0, ov)
            bt = jnp.where(match, t, bt)
        ov_ref[...] = ov
        bt_ref[...] = bt
        return carry

    jax.lax.fori_loop(0, _MP // _UNROLL, pass2, 0)

    def gather(i, carry):
        bt = bt_ref[...]
        c0 = c0_ref[...]
        c1 = c1_ref[...]
        c2 = c2_ref[...]
        c3 = c3_ref[...]
        for j in range(_UNROLL):
            t = _UNROLL * i + j
            eq = bt == t
            c0 = jnp.where(eq, tgt_ref[t, 6][:, None, :], c0)
            c1 = jnp.where(eq, tgt_ref[t, 7][:, None, :], c1)
            c2 = jnp.where(eq, tgt_ref[t, 8][:, None, :], c2)
            c3 = jnp.where(eq, tgt_ref[t, 9][:, None, :], c3)
        c0_ref[...] = c0
        c1_ref[...] = c1
        c2_ref[...] = c2
        c3_ref[...] = c3
        return carry

    zero = jnp.zeros(shp, f32)
    c0_ref[...] = zero
    c1_ref[...] = zero
    c2_ref[...] = zero
    c3_ref[...] = zero
    jax.lax.fori_loop(0, _MP // _UNROLL, gather, 0)

    lane_valid = lin_ref[...][None] < _P
    pos = jnp.logical_and(ov_ref[...] >= _THRESH, lane_valid)

    pcx = pr_ref[5][None]
    pcy = pr_ref[6][None]
    ivw = pr_ref[7][None]
    ivh = pr_ref[8][None]
    lwp = pr_ref[9][None]
    lhp = pr_ref[10][None]
    gcx = (c0_ref[...] - pcx) * ivw
    gcy = (c1_ref[...] - pcy) * ivh
    gw = c2_ref[...] + lwp
    gh = c3_ref[...] + lhp

    def _sl1(d):
        ad = jnp.abs(d)
        return jnp.where(ad < 1.0, 0.5 * ad * ad, ad - 0.5)

    sl1 = (_sl1(loc_ref[0] - gcx) + _sl1(loc_ref[1] - gcy)
           + _sl1(loc_ref[2] - gw) + _sl1(loc_ref[3] - gh))
    loss_l = jnp.sum(jnp.where(pos, sl1, 0.0), axis=(1, 2), keepdims=True)

    ca = conf_ref[0]
    cb = conf_ref[1]
    mx = jnp.maximum(ca, cb)
    lse = mx + jnp.log(jnp.exp(ca - mx) + jnp.exp(cb - mx))
    gath = jnp.where(pos, cb, ca)
    lc = lse - gath
    lc_pos = jnp.sum(jnp.where(pos, lc, 0.0), axis=(1, 2), keepdims=True)

    mine = jnp.where(jnp.logical_or(pos, jnp.logical_not(lane_valid)), 0.0, lc)
    mine = jnp.maximum(mine, 0.0)
    bits = jnp.maximum(jax.lax.bitcast_convert_type(mine, jnp.int32), 0)

    npos = jnp.sum(pos.astype(jnp.int32), axis=(1, 2), keepdims=True)
    k = jnp.minimum(_NEGPOS * npos, _P - 1)

    def bstep(i, pfx):
        cand = pfx | jnp.left_shift(jnp.int32(1), 30 - i)
        cnt = jnp.sum((bits >= cand).astype(jnp.int32), axis=(1, 2),
                      keepdims=True)
        return jnp.where(cnt >= k, cand, pfx)

    tbits = jax.lax.fori_loop(0, 31, bstep, jnp.zeros((_B, 1, 1), jnp.int32))
    gt = bits > tbits
    cnt_gt = jnp.sum(gt.astype(jnp.int32), axis=(1, 2), keepdims=True)
    sum_gt = jnp.sum(jnp.where(gt, mine, 0.0), axis=(1, 2), keepdims=True)
    tval = jax.lax.bitcast_convert_type(tbits, f32)
    topk = sum_gt + (k - cnt_gt).astype(f32) * tval
    loss_c = lc_pos + jnp.where(k > 0, topk, 0.0)

    lane = jax.lax.broadcasted_iota(jnp.int32, (_B, _LANES), 1)
    llb = jnp.broadcast_to(loss_l[:, :, 0], (_B, _LANES))
    lcb = jnp.broadcast_to(loss_c[:, :, 0], (_B, _LANES))
    npb = jnp.broadcast_to(npos[:, :, 0].astype(f32), (_B, _LANES))
    out_ref[...] = jnp.where(lane == 0, llb,
                             jnp.where(lane == 1, lcb,
                                       jnp.where(lane == 2, npb, 0.0)))


def kernel(loc_data, conf_data, priors, targets):
    B = loc_data.shape[0]
    w = priors[:, 2]
    h = priors[:, 3]
    pf1 = priors[:, 0] - w * 0.5
    pf2 = priors[:, 1] - h * 0.5
    pf3 = priors[:, 0] + w * 0.5
    pf4 = priors[:, 1] + h * 0.5
    parea = (pf3 - pf1) * (pf4 - pf2)
    ch = jnp.stack([
        pf1, pf2, pf3, pf4, parea,
        priors[:, 0], priors[:, 1],
        1.0 / (0.1 * w), 1.0 / (0.1 * h),
        -5.0 * jnp.log(w), -5.0 * jnp.log(h),
    ], axis=0)
    pad = _PPAD - _P
    padvals = jnp.array([2.0, 2.0, 2.0, 2.0, 0.0, 0.0, 0.0, 1.0, 1.0, 0.0,
                         0.0], dtype=jnp.float32)
    ch = jnp.concatenate([ch, jnp.broadcast_to(padvals[:, None], (11, pad))],
                         axis=1)
    ch = ch.reshape(11, _ROWS, _LANES)
    lin = jnp.arange(_PPAD, dtype=jnp.int32).reshape(_ROWS, _LANES)

    loc_r = jnp.pad(jnp.transpose(loc_data, (2, 0, 1)), ((0, 0), (0, 0), (0, pad)))
    loc_r = loc_r.reshape(4, B, _ROWS, _LANES)
    conf_r = jnp.pad(jnp.transpose(conf_data, (2, 0, 1)), ((0, 0), (0, 0), (0, pad)))
    conf_r = conf_r.reshape(2, B, _ROWS, _LANES)

    validf = (jnp.sum(targets, axis=2) > 0.0).astype(jnp.float32)
    boxes = jnp.where(validf[:, :, None] > 0, targets, 3.0)
    tarea = (boxes[:, :, 2] - boxes[:, :, 0]) * (boxes[:, :, 3] - boxes[:, :, 1])
    tcx = (boxes[:, :, 0] + boxes[:, :, 2]) * 0.5
    tcy = (boxes[:, :, 1] + boxes[:, :, 3]) * 0.5
    ltw = 5.0 * jnp.log(jnp.maximum(boxes[:, :, 2] - boxes[:, :, 0], 1e-6))
    lth = 5.0 * jnp.log(jnp.maximum(boxes[:, :, 3] - boxes[:, :, 1], 1e-6))
    tchan = jnp.concatenate(
        [boxes, validf[:, :, None], tarea[:, :, None], tcx[:, :, None],
         tcy[:, :, None], ltw[:, :, None], lth[:, :, None]], axis=2)
    padrow = jnp.array([3.0, 3.0, 3.0, 3.0, 0.0, 0.0, 3.0, 3.0,
                        5.0 * jnp.log(1e-6), 5.0 * jnp.log(1e-6)],
                       dtype=jnp.float32)
    tchan = jnp.concatenate(
        [tchan, jnp.broadcast_to(padrow[None, None, :], (B, _MP - _M, 10))],
        axis=1)
    tgt = jnp.broadcast_to(jnp.transpose(tchan, (1, 2, 0))[:, :, :, None],
                           (_MP, 10, B, _LANES))

    out = pl.pallas_call(
        _body,
        out_shape=jax.ShapeDtypeStruct((B, _LANES), jnp.float32),
        scratch_shapes=[
            pltpu.VMEM((B, _ROWS, _LANES), jnp.float32),
            pltpu.VMEM((B, _ROWS, _LANES), jnp.int32),
            pltpu.VMEM((B, _ROWS, _LANES), jnp.float32),
            pltpu.VMEM((B, _ROWS, _LANES), jnp.float32),
            pltpu.VMEM((B, _ROWS, _LANES), jnp.float32),
            pltpu.VMEM((B, _ROWS, _LANES), jnp.float32),
            pltpu.VMEM((_MP, B, _LANES), jnp.int32),
        ],
    )(tgt, ch, lin, loc_r, conf_r)

    loss_l = jnp.sum(out[:, 0])
    loss_c = jnp.sum(out[:, 1])
    n = jnp.maximum(jnp.sum(out[:, 2]), 1.0)
    return loss_l / n, loss_c / n


# speedup vs baseline: 1.2848x; 1.0129x over previous
import jax
import jax.numpy as jnp
from jax.experimental import pallas as pl
from jax.experimental.pallas import tpu as pltpu

_P = 8732
_M = 50
_MP = 52
_B = 32
_ROWS = 72
_LANES = 128
_PPAD = _ROWS * _LANES
_THRESH = 0.5
_NEGPOS = 3
_UNROLL = 4
_BG = 16


def _body(tgt_ref, pr_ref, lin_ref, loc_ref, conf_ref, out_ref,
          ov_ref, bt_ref, c0_ref, c1_ref, c2_ref, c3_ref, bp_ref):
    f32 = jnp.float32
    shp = (_BG, _ROWS, _LANES)
    px1 = pr_ref[0][None]
    py1 = pr_ref[1][None]
    px2 = pr_ref[2][None]
    py2 = pr_ref[3][None]
    parea = pr_ref[4][None]

    ov_ref[...] = jnp.full(shp, -2.0, f32)
    bt_ref[...] = jnp.zeros(shp, jnp.int32)

    def pass1(i, carry):
        ov = ov_ref[...]
        bt = bt_ref[...]
        lin = lin_ref[...][None]
        for j in range(_UNROLL):
            t = _UNROLL * i + j
            tx1 = tgt_ref[t, 0][:, None, :]
            ty1 = tgt_ref[t, 1][:, None, :]
            tx2 = tgt_ref[t, 2][:, None, :]
            ty2 = tgt_ref[t, 3][:, None, :]
            ix = jnp.maximum(jnp.minimum(tx2, px2) - jnp.maximum(tx1, px1), 0.0)
            iy = jnp.maximum(jnp.minimum(ty2, py2) - jnp.maximum(ty1, py1), 0.0)
            inter = ix * iy
            tarea = tgt_ref[t, 5][:, None, :]
            union = jnp.maximum(tarea + parea - inter, 1e-10)
            iou = inter / union
            m = jnp.max(iou, axis=(1, 2), keepdims=True)
            bp = jnp.min(jnp.where(iou == m, lin, jnp.int32(2**30)),
                         axis=(1, 2), keepdims=True)
            validb = tgt_ref[t, 4][:, None, 0:1] > 0.0
            bp = jnp.where(validb, bp, -1)
            bp_ref[t] = jnp.broadcast_to(bp[:, 0, :], (_BG, _LANES))
            better = iou > ov
            ov = jnp.where(better, iou, ov)
            bt = jnp.where(better, t, bt)
        ov_ref[...] = ov
        bt_ref[...] = bt
        return carry

    jax.lax.fori_loop(0, _MP // _UNROLL, pass1, 0)

    def pass2(i, carry):
        ov = ov_ref[...]
        bt = bt_ref[...]
        lin = lin_ref[...][None]
        for j in range(_UNROLL):
            t = (_MP - 1) - (_UNROLL * i + j)
            match = lin == bp_ref[t][:, None, :]
            ov = jnp.where(match, 2.0, ov)
            bt = jnp.where(match, t, bt)
        ov_ref[...] = ov
        bt_ref[...] = bt
        return carry

    jax.lax.fori_loop(0, _MP // _UNROLL, pass2, 0)

    def gather(i, carry):
        bt = bt_ref[...]
        c0 = c0_ref[...]
        c1 = c1_ref[...]
        c2 = c2_ref[...]
        c3 = c3_ref[...]
        for j in range(_UNROLL):
            t = _UNROLL * i + j
            eq = bt == t
            c0 = jnp.where(eq, tgt_ref[t, 6][:, None, :], c0)
            c1 = jnp.where(eq, tgt_ref[t, 7][:, None, :], c1)
            c2 = jnp.where(eq, tgt_ref[t, 8][:, None, :], c2)
            c3 = jnp.where(eq, tgt_ref[t, 9][:, None, :], c3)
        c0_ref[...] = c0
        c1_ref[...] = c1
        c2_ref[...] = c2
        c3_ref[...] = c3
        return carry

    zero = jnp.zeros(shp, f32)
    c0_ref[...] = zero
    c1_ref[...] = zero
    c2_ref[...] = zero
    c3_ref[...] = zero
    jax.lax.fori_loop(0, _MP // _UNROLL, gather, 0)

    lane_valid = lin_ref[...][None] < _P
    pos = jnp.logical_and(ov_ref[...] >= _THRESH, lane_valid)

    pcx = pr_ref[5][None]
    pcy = pr_ref[6][None]
    ivw = pr_ref[7][None]
    ivh = pr_ref[8][None]
    lwp = pr_ref[9][None]
    lhp = pr_ref[10][None]
    gcx = (c0_ref[...] - pcx) * ivw
    gcy = (c1_ref[...] - pcy) * ivh
    gw = c2_ref[...] + lwp
    gh = c3_ref[...] + lhp

    def _sl1(d):
        ad = jnp.abs(d)
        return jnp.where(ad < 1.0, 0.5 * ad * ad, ad - 0.5)

    sl1 = (_sl1(loc_ref[0] - gcx) + _sl1(loc_ref[1] - gcy)
           + _sl1(loc_ref[2] - gw) + _sl1(loc_ref[3] - gh))
    loss_l = jnp.sum(jnp.where(pos, sl1, 0.0), axis=(1, 2), keepdims=True)

    ca = conf_ref[0]
    cb = conf_ref[1]
    mx = jnp.maximum(ca, cb)
    lse = mx + jnp.log(jnp.exp(ca - mx) + jnp.exp(cb - mx))
    gath = jnp.where(pos, cb, ca)
    lc = lse - gath
    lc_pos = jnp.sum(jnp.where(pos, lc, 0.0), axis=(1, 2), keepdims=True)

    mine = jnp.where(jnp.logical_or(pos, jnp.logical_not(lane_valid)), 0.0, lc)
    mine = jnp.maximum(mine, 0.0)
    bits = jnp.maximum(jax.lax.bitcast_convert_type(mine, jnp.int32), 0)

    npos = jnp.sum(pos.astype(jnp.int32), axis=(1, 2), keepdims=True)
    k = jnp.minimum(_NEGPOS * npos, _P - 1)

    def bstep(i, pfx):
        cand = pfx | jnp.left_shift(jnp.int32(1), 30 - i)
        cnt = jnp.sum((bits >= cand).astype(jnp.int32), axis=(1, 2),
                      keepdims=True)
        return jnp.where(cnt >= k, cand, pfx)

    tbits = jax.lax.fori_loop(0, 31, bstep, jnp.zeros((_BG, 1, 1), jnp.int32))
    gt = bits > tbits
    cnt_gt = jnp.sum(gt.astype(jnp.int32), axis=(1, 2), keepdims=True)
    sum_gt = jnp.sum(jnp.where(gt, mine, 0.0), axis=(1, 2), keepdims=True)
    tval = jax.lax.bitcast_convert_type(tbits, f32)
    topk = sum_gt + (k - cnt_gt).astype(f32) * tval
    loss_c = lc_pos + jnp.where(k > 0, topk, 0.0)

    lane = jax.lax.broadcasted_iota(jnp.int32, (_BG, _LANES), 1)
    llb = jnp.broadcast_to(loss_l[:, :, 0], (_BG, _LANES))
    lcb = jnp.broadcast_to(loss_c[:, :, 0], (_BG, _LANES))
    npb = jnp.broadcast_to(npos[:, :, 0].astype(f32), (_BG, _LANES))
    out_ref[...] = jnp.where(lane == 0, llb,
                             jnp.where(lane == 1, lcb,
                                       jnp.where(lane == 2, npb, 0.0)))


def kernel(loc_data, conf_data, priors, targets):
    B = loc_data.shape[0]
    w = priors[:, 2]
    h = priors[:, 3]
    pf1 = priors[:, 0] - w * 0.5
    pf2 = priors[:, 1] - h * 0.5
    pf3 = priors[:, 0] + w * 0.5
    pf4 = priors[:, 1] + h * 0.5
    parea = (pf3 - pf1) * (pf4 - pf2)
    ch = jnp.stack([
        pf1, pf2, pf3, pf4, parea,
        priors[:, 0], priors[:, 1],
        1.0 / (0.1 * w), 1.0 / (0.1 * h),
        -5.0 * jnp.log(w), -5.0 * jnp.log(h),
    ], axis=0)
    pad = _PPAD - _P
    padvals = jnp.array([2.0, 2.0, 2.0, 2.0, 0.0, 0.0, 0.0, 1.0, 1.0, 0.0,
                         0.0], dtype=jnp.float32)
    ch = jnp.concatenate([ch, jnp.broadcast_to(padvals[:, None], (11, pad))],
                         axis=1)
    ch = ch.reshape(11, _ROWS, _LANES)
    lin = jnp.arange(_PPAD, dtype=jnp.int32).reshape(_ROWS, _LANES)

    loc_r = jnp.pad(jnp.transpose(loc_data, (2, 0, 1)), ((0, 0), (0, 0), (0, pad)))
    loc_r = loc_r.reshape(4, B, _ROWS, _LANES)
    conf_r = jnp.pad(jnp.transpose(conf_data, (2, 0, 1)), ((0, 0), (0, 0), (0, pad)))
    conf_r = conf_r.reshape(2, B, _ROWS, _LANES)

    validf = (jnp.sum(targets, axis=2) > 0.0).astype(jnp.float32)
    boxes = jnp.where(validf[:, :, None] > 0, targets, 3.0)
    tarea = (boxes[:, :, 2] - boxes[:, :, 0]) * (boxes[:, :, 3] - boxes[:, :, 1])
    tcx = (boxes[:, :, 0] + boxes[:, :, 2]) * 0.5
    tcy = (boxes[:, :, 1] + boxes[:, :, 3]) * 0.5
    ltw = 5.0 * jnp.log(jnp.maximum(boxes[:, :, 2] - boxes[:, :, 0], 1e-6))
    lth = 5.0 * jnp.log(jnp.maximum(boxes[:, :, 3] - boxes[:, :, 1], 1e-6))
    tchan = jnp.concatenate(
        [boxes, validf[:, :, None], tarea[:, :, None], tcx[:, :, None],
         tcy[:, :, None], ltw[:, :, None], lth[:, :, None]], axis=2)
    padrow = jnp.array([3.0, 3.0, 3.0, 3.0, 0.0, 0.0, 3.0, 3.0,
                        5.0 * jnp.log(1e-6), 5.0 * jnp.log(1e-6)],
                       dtype=jnp.float32)
    tchan = jnp.concatenate(
        [tchan, jnp.broadcast_to(padrow[None, None, :], (B, _MP - _M, 10))],
        axis=1)
    tgt = jnp.broadcast_to(jnp.transpose(tchan, (1, 2, 0))[:, :, :, None],
                           (_MP, 10, B, _LANES))

    out = pl.pallas_call(
        _body,
        grid=(B // _BG,),
        in_specs=[
            pl.BlockSpec((_MP, 10, _BG, _LANES), lambda g: (0, 0, g, 0)),
            pl.BlockSpec((11, _ROWS, _LANES), lambda g: (0, 0, 0)),
            pl.BlockSpec((_ROWS, _LANES), lambda g: (0, 0)),
            pl.BlockSpec((4, _BG, _ROWS, _LANES), lambda g: (0, g, 0, 0)),
            pl.BlockSpec((2, _BG, _ROWS, _LANES), lambda g: (0, g, 0, 0)),
        ],
        out_specs=pl.BlockSpec((_BG, _LANES), lambda g: (g, 0)),
        out_shape=jax.ShapeDtypeStruct((B, _LANES), jnp.float32),
        scratch_shapes=[
            pltpu.VMEM((_BG, _ROWS, _LANES), jnp.float32),
            pltpu.VMEM((_BG, _ROWS, _LANES), jnp.int32),
            pltpu.VMEM((_BG, _ROWS, _LANES), jnp.float32),
            pltpu.VMEM((_BG, _ROWS, _LANES), jnp.float32),
            pltpu.VMEM((_BG, _ROWS, _LANES), jnp.float32),
            pltpu.VMEM((_BG, _ROWS, _LANES), jnp.float32),
            pltpu.VMEM((_MP, _BG, _LANES), jnp.int32),
        ],
    )(tgt, ch, lin, loc_r, conf_r)

    loss_l = jnp.sum(out[:, 0])
    loss_c = jnp.sum(out[:, 1])
    n = jnp.maximum(jnp.sum(out[:, 2]), 1.0)
    return loss_l / n, loss_c / n


# speedup vs baseline: 1.3150x; 1.0235x over previous
import jax
import jax.numpy as jnp
from jax.experimental import pallas as pl
from jax.experimental.pallas import tpu as pltpu

_P = 8732
_M = 50
_MP = 52
_B = 32
_ROWS = 72
_LANES = 128
_PPAD = _ROWS * _LANES
_THRESH = 0.5
_NEGPOS = 3
_UNROLL = 4


def _body(tgt_ref, pr_ref, lin_ref, loc_ref, conf_ref, out_ref,
          ov_ref, bt_ref, c0_ref, c1_ref, c2_ref, c3_ref, bp_ref):
    f32 = jnp.float32
    shp = (_B, _ROWS, _LANES)
    px1 = pr_ref[0][None]
    py1 = pr_ref[1][None]
    px2 = pr_ref[2][None]
    py2 = pr_ref[3][None]
    parea = pr_ref[4][None]

    ov_ref[...] = jnp.full(shp, -2.0, f32)
    bt_ref[...] = jnp.zeros(shp, jnp.int32)

    def pass1(i, carry):
        ov = ov_ref[...]
        bt = bt_ref[...]
        lin = lin_ref[...][None]
        for j in range(_UNROLL):
            t = _UNROLL * i + j
            tx1 = tgt_ref[t, 0][:, None, :]
            ty1 = tgt_ref[t, 1][:, None, :]
            tx2 = tgt_ref[t, 2][:, None, :]
            ty2 = tgt_ref[t, 3][:, None, :]
            ix = jnp.maximum(jnp.minimum(tx2, px2) - jnp.maximum(tx1, px1), 0.0)
            iy = jnp.maximum(jnp.minimum(ty2, py2) - jnp.maximum(ty1, py1), 0.0)
            inter = ix * iy
            tarea = tgt_ref[t, 5][:, None, :]
            union = jnp.maximum(tarea + parea - inter, 1e-10)
            iou = inter / union
            m = jnp.max(iou, axis=(1, 2), keepdims=True)
            bp = jnp.min(jnp.where(iou == m, lin, jnp.int32(2**30)),
                         axis=(1, 2), keepdims=True)
            validb = tgt_ref[t, 4][:, None, 0:1] > 0.0
            bp = jnp.where(validb, bp, -1)
            bp_ref[t] = jnp.broadcast_to(bp[:, 0, :], (_B, _LANES))
            better = iou > ov
            ov = jnp.where(better, iou, ov)
            bt = jnp.where(better, t, bt)
        ov_ref[...] = ov
        bt_ref[...] = bt
        return carry

    jax.lax.fori_loop(0, _MP // _UNROLL, pass1, 0)

    def pass2(i, carry):
        ov = ov_ref[...]
        bt = bt_ref[...]
        lin = lin_ref[...][None]
        for j in range(_UNROLL):
            t = (_MP - 1) - (_UNROLL * i + j)
            match = lin == bp_ref[t][:, None, :]
            ov = jnp.where(match, 2.0, ov)
            bt = jnp.where(match, t, bt)
        ov_ref[...] = ov
        bt_ref[...] = bt
        return carry

    jax.lax.fori_loop(0, _MP // _UNROLL, pass2, 0)

    def gather(i, carry):
        bt = bt_ref[...]
        c0 = c0_ref[...]
        c1 = c1_ref[...]
        c2 = c2_ref[...]
        c3 = c3_ref[...]
        for j in range(_UNROLL):
            t = _UNROLL * i + j
            eq = bt == t
            c0 = jnp.where(eq, tgt_ref[t, 6][:, None, :], c0)
            c1 = jnp.where(eq, tgt_ref[t, 7][:, None, :], c1)
            c2 = jnp.where(eq, tgt_ref[t, 8][:, None, :], c2)
            c3 = jnp.where(eq, tgt_ref[t, 9][:, None, :], c3)
        c0_ref[...] = c0
        c1_ref[...] = c1
        c2_ref[...] = c2
        c3_ref[...] = c3
        return carry

    zero = jnp.zeros(shp, f32)
    c0_ref[...] = zero
    c1_ref[...] = zero
    c2_ref[...] = zero
    c3_ref[...] = zero
    jax.lax.fori_loop(0, _MP // _UNROLL, gather, 0)

    lane_valid = lin_ref[...][None] < _P
    pos = jnp.logical_and(ov_ref[...] >= _THRESH, lane_valid)

    pcx = pr_ref[5][None]
    pcy = pr_ref[6][None]
    ivw = pr_ref[7][None]
    ivh = pr_ref[8][None]
    lwp = pr_ref[9][None]
    lhp = pr_ref[10][None]
    gcx = (c0_ref[...] - pcx) * ivw
    gcy = (c1_ref[...] - pcy) * ivh
    gw = c2_ref[...] + lwp
    gh = c3_ref[...] + lhp

    def _sl1(d):
        ad = jnp.abs(d)
        return jnp.where(ad < 1.0, 0.5 * ad * ad, ad - 0.5)

    sl1 = (_sl1(loc_ref[0] - gcx) + _sl1(loc_ref[1] - gcy)
           + _sl1(loc_ref[2] - gw) + _sl1(loc_ref[3] - gh))
    loss_l = jnp.sum(jnp.where(pos, sl1, 0.0), axis=(1, 2), keepdims=True)

    ca = conf_ref[0]
    cb = conf_ref[1]
    mx = jnp.maximum(ca, cb)
    lse = mx + jnp.log(jnp.exp(ca - mx) + jnp.exp(cb - mx))
    gath = jnp.where(pos, cb, ca)
    lc = lse - gath
    lc_pos = jnp.sum(jnp.where(pos, lc, 0.0), axis=(1, 2), keepdims=True)

    mine = jnp.where(jnp.logical_or(pos, jnp.logical_not(lane_valid)), 0.0, lc)
    mine = jnp.maximum(mine, 0.0)
    bits = jnp.maximum(jax.lax.bitcast_convert_type(mine, jnp.int32), 0)

    npos = jnp.sum(pos.astype(jnp.int32), axis=(1, 2), keepdims=True)
    k = jnp.minimum(_NEGPOS * npos, _P - 1)

    def bstep(i, pfx):
        cand = pfx | jnp.left_shift(jnp.int32(1), 30 - i)
        cnt = jnp.sum((bits >= cand).astype(jnp.int32), axis=(1, 2),
                      keepdims=True)
        return jnp.where(cnt >= k, cand, pfx)

    tbits = jax.lax.fori_loop(0, 31, bstep, jnp.zeros((_B, 1, 1), jnp.int32))
    gt = bits > tbits
    cnt_gt = jnp.sum(gt.astype(jnp.int32), axis=(1, 2), keepdims=True)
    sum_gt = jnp.sum(jnp.where(gt, mine, 0.0), axis=(1, 2), keepdims=True)
    tval = jax.lax.bitcast_convert_type(tbits, f32)
    topk = sum_gt + (k - cnt_gt).astype(f32) * tval
    loss_c = lc_pos + jnp.where(k > 0, topk, 0.0)

    lane = jax.lax.broadcasted_iota(jnp.int32, (_B, _LANES), 1)
    llb = jnp.broadcast_to(loss_l[:, :, 0], (_B, _LANES))
    lcb = jnp.broadcast_to(loss_c[:, :, 0], (_B, _LANES))
    npb = jnp.broadcast_to(npos[:, :, 0].astype(f32), (_B, _LANES))
    out_ref[...] = jnp.where(lane == 0, llb,
                             jnp.where(lane == 1, lcb,
                                       jnp.where(lane == 2, npb, 0.0)))


def kernel(loc_data, conf_data, priors, targets):
    B = loc_data.shape[0]
    w = priors[:, 2]
    h = priors[:, 3]
    pf1 = priors[:, 0] - w * 0.5
    pf2 = priors[:, 1] - h * 0.5
    pf3 = priors[:, 0] + w * 0.5
    pf4 = priors[:, 1] + h * 0.5
    parea = (pf3 - pf1) * (pf4 - pf2)
    ch = jnp.stack([
        pf1, pf2, pf3, pf4, parea,
        priors[:, 0], priors[:, 1],
        1.0 / (0.1 * w), 1.0 / (0.1 * h),
        -5.0 * jnp.log(w), -5.0 * jnp.log(h),
    ], axis=0)
    pad = _PPAD - _P
    padvals = jnp.array([2.0, 2.0, 2.0, 2.0, 0.0, 0.0, 0.0, 1.0, 1.0, 0.0,
                         0.0], dtype=jnp.float32)
    ch = jnp.concatenate([ch, jnp.broadcast_to(padvals[:, None], (11, pad))],
                         axis=1)
    ch = ch.reshape(11, _ROWS, _LANES)
    lin = jnp.arange(_PPAD, dtype=jnp.int32).reshape(_ROWS, _LANES)

    loc_r = jnp.pad(jnp.transpose(loc_data, (2, 0, 1)), ((0, 0), (0, 0), (0, pad)))
    loc_r = loc_r.reshape(4, B, _ROWS, _LANES)
    conf_r = jnp.pad(jnp.transpose(conf_data, (2, 0, 1)), ((0, 0), (0, 0), (0, pad)))
    conf_r = conf_r.reshape(2, B, _ROWS, _LANES)

    validf = (jnp.sum(targets, axis=2) > 0.0).astype(jnp.float32)
    boxes = jnp.where(validf[:, :, None] > 0, targets, 3.0)
    tarea = (boxes[:, :, 2] - boxes[:, :, 0]) * (boxes[:, :, 3] - boxes[:, :, 1])
    tcx = (boxes[:, :, 0] + boxes[:, :, 2]) * 0.5
    tcy = (boxes[:, :, 1] + boxes[:, :, 3]) * 0.5
    ltw = 5.0 * jnp.log(jnp.maximum(boxes[:, :, 2] - boxes[:, :, 0], 1e-6))
    lth = 5.0 * jnp.log(jnp.maximum(boxes[:, :, 3] - boxes[:, :, 1], 1e-6))
    tchan = jnp.concatenate(
        [boxes, validf[:, :, None], tarea[:, :, None], tcx[:, :, None],
         tcy[:, :, None], ltw[:, :, None], lth[:, :, None]], axis=2)
    padrow = jnp.array([3.0, 3.0, 3.0, 3.0, 0.0, 0.0, 3.0, 3.0,
                        5.0 * jnp.log(1e-6), 5.0 * jnp.log(1e-6)],
                       dtype=jnp.float32)
    tchan = jnp.concatenate(
        [tchan, jnp.broadcast_to(padrow[None, None, :], (B, _MP - _M, 10))],
        axis=1)
    tgt = jnp.broadcast_to(jnp.transpose(tchan, (1, 2, 0))[:, :, :, None],
                           (_MP, 10, B, _LANES))

    out = pl.pallas_call(
        _body,
        out_shape=jax.ShapeDtypeStruct((B, _LANES), jnp.float32),
        scratch_shapes=[
            pltpu.VMEM((B, _ROWS, _LANES), jnp.float32),
            pltpu.VMEM((B, _ROWS, _LANES), jnp.int32),
            pltpu.VMEM((B, _ROWS, _LANES), jnp.float32),
            pltpu.VMEM((B, _ROWS, _LANES), jnp.float32),
            pltpu.VMEM((B, _ROWS, _LANES), jnp.float32),
            pltpu.VMEM((B, _ROWS, _LANES), jnp.float32),
            pltpu.VMEM((_MP, B, _LANES), jnp.int32),
        ],
    )(tgt, ch, lin, loc_r, conf_r)

    loss_l = jnp.sum(out[:, 0])
    loss_c = jnp.sum(out[:, 1])
    n = jnp.maximum(jnp.sum(out[:, 2]), 1.0)
    return loss_l / n, loss_c / n
